# Initial kernel scaffold; baseline (speedup 1.0000x reference)
import jax
import jax.numpy as jnp
from jax.experimental import pallas as pl


def kernel(xyz, child_xyz, feats):
    raise NotImplementedError("write your pallas kernel here")


# speedup vs baseline: 376.6338x; 376.6338x over previous
import functools

import numpy as np
import jax
import jax.numpy as jnp
from jax import lax
from jax.experimental import pallas as pl
from jax.experimental.pallas import tpu as pltpu
from jax.experimental.pallas import tpu_sc as plsc

BS = 4
N = 8192
M = 2048
C = 128
COUT = C + 3
K = 32
MW = 256
NGROUP = MW // 16
CJ = 32
NBUF = 4
R2 = np.float32(0.2 * 0.2)


def _sc_query_group(xyzc, childc, feats):
    mesh = plsc.VectorSubcoreMesh(core_axis_name="c", subcore_axis_name="s")

    @functools.partial(
        pl.kernel,
        out_type=(
            jax.ShapeDtypeStruct((BS * COUT * M * K,), jnp.float32),
            jax.ShapeDtypeStruct((BS * M * K,), jnp.int32),
        ),
        mesh=mesh,
        compiler_params=pltpu.CompilerParams(needs_layout_passes=False),
        scratch_types=[
            pltpu.VMEM((N,), jnp.float32),
            pltpu.VMEM((N,), jnp.float32),
            pltpu.VMEM((N,), jnp.float32),
            pltpu.VMEM((MW,), jnp.float32),
            pltpu.VMEM((MW,), jnp.float32),
            pltpu.VMEM((MW,), jnp.float32),
            pltpu.VMEM((MW * K,), jnp.int32),
            pltpu.VMEM((MW * K,), jnp.int32),
            pltpu.VMEM((NBUF * N,), jnp.float32),
            pltpu.VMEM((NBUF * MW * K,), jnp.float32),
            [pltpu.SemaphoreType.DMA] * NBUF,
            [pltpu.SemaphoreType.DMA] * NBUF,
        ],
    )
    def body(xyzc_ref, childc_ref, feats_ref, grouped_ref, filled_ref,
             px, py, pz, cx, cy, cz, idxb, fillb, tbl, ob, sem_t, sem_o):
        wid = lax.axis_index("s") * 2 + lax.axis_index("c")
        b = wid // 8
        mbase = (wid % 8) * MW
        obase = mbase * K

        pltpu.sync_copy(xyzc_ref.at[pl.ds((b * 3 + 0) * N, N)], px)
        pltpu.sync_copy(xyzc_ref.at[pl.ds((b * 3 + 1) * N, N)], py)
        pltpu.sync_copy(xyzc_ref.at[pl.ds((b * 3 + 2) * N, N)], pz)
        pltpu.sync_copy(childc_ref.at[pl.ds((b * 3 + 0) * M + mbase, MW)], cx)
        pltpu.sync_copy(childc_ref.at[pl.ds((b * 3 + 1) * M + mbase, MW)], cy)
        pltpu.sync_copy(childc_ref.at[pl.ds((b * 3 + 2) * M + mbase, MW)], cz)

        iota16 = lax.iota(jnp.int32, 16)
        zeros16 = jnp.zeros((16,), jnp.int32)

        def zstep(v, _):
            idxb[pl.ds(v * 16, 16)] = zeros16
            return 0
        lax.fori_loop(0, MW * K // 16, zstep, 0)

        def group_body(g, _):
            base = g * 16
            cxv = cx[pl.ds(base, 16)]
            cyv = cy[pl.ds(base, 16)]
            czv = cz[pl.ds(base, 16)]
            rowbase = (base + iota16) * K

            def ocond(carry):
                j0, ptrv, done = carry
                return jnp.logical_and(j0 < N, jnp.logical_not(done))

            def obody(carry):
                j0, ptrv0, _ = carry

                def istep(t, ptrv):
                    jv = jnp.full((16,), j0 + t, jnp.int32)
                    xj = plsc.load_gather(px, [jv])
                    yj = plsc.load_gather(py, [jv])
                    zj = plsc.load_gather(pz, [jv])
                    dx = cxv - xj
                    dy = cyv - yj
                    dz = czv - zj
                    d2 = (dx * dx + dy * dy) + dz * dz
                    msk = d2 <= R2
                    okm = jnp.logical_and(msk, ptrv < K)
                    plsc.store_scatter(idxb, [rowbase + ptrv], jv, mask=okm)
                    return ptrv + msk.astype(jnp.int32)

                ptrv1 = lax.fori_loop(0, CJ, istep, ptrv0, unroll=4)
                ndone = jnp.sum((ptrv1 >= K).astype(jnp.int32))
                return (j0 + CJ, ptrv1, ndone >= 16)

            _, ptrv, _ = lax.while_loop(
                ocond, obody, (jnp.int32(0), zeros16, jnp.bool_(False)))
            cnt = jnp.minimum(ptrv, K)

            def fstep(s, _):
                plsc.store_scatter(fillb, [rowbase + s],
                                   (cnt > s).astype(jnp.int32))
                return 0
            lax.fori_loop(0, K, fstep, 0)
            return 0
        lax.fori_loop(0, NGROUP, group_body, 0)

        pltpu.sync_copy(fillb, filled_ref.at[pl.ds(b * M * K + obase, MW * K)])

        NV = MW * K // 16

        for ch, (src, cref) in enumerate(((px, cx), (py, cy), (pz, cz))):
            def xstep(v, _, src=src, cref=cref):
                idxv = idxb[pl.ds(v * 16, 16)]
                cv = plsc.load_gather(cref, [jnp.full((16,), v // 2, jnp.int32)])
                ob[pl.ds(v * 16, 16)] = plsc.load_gather(src, [idxv]) - cv
                return 0
            lax.fori_loop(0, NV, xstep, 0, unroll=2)
            pltpu.sync_copy(
                ob.at[pl.ds(0, MW * K)],
                grouped_ref.at[pl.ds((b * COUT + ch) * M * K + obase, MW * K)])

        def gslice(ch):
            return grouped_ref.at[pl.ds((b * COUT + ch) * M * K + obase, MW * K)]

        for kb in range(NBUF):
            pltpu.async_copy(feats_ref.at[pl.ds((b * C + kb) * N, N)],
                             tbl.at[pl.ds(kb * N, N)], sem_t[kb])

        def fgroup(i, _):
            for kb in range(NBUF):
                cf = i * NBUF + kb
                toff = kb * N
                ooff = kb * MW * K
                pltpu.make_async_copy(feats_ref.at[pl.ds((b * C + cf) * N, N)],
                                      tbl.at[pl.ds(toff, N)], sem_t[kb]).wait()

                @pl.when(cf >= NBUF)
                def _():
                    pltpu.make_async_copy(
                        ob.at[pl.ds(ooff, MW * K)],
                        gslice(3 + cf - NBUF),
                        sem_o[kb]).wait()

                def gstep(v, _, toff=toff, ooff=ooff):
                    idxv = idxb[pl.ds(v * 16, 16)]
                    ob[pl.ds(ooff + v * 16, 16)] = plsc.load_gather(
                        tbl, [idxv + toff])
                    return 0
                lax.fori_loop(0, NV, gstep, 0, unroll=4)

                @pl.when(cf + NBUF < C)
                def _():
                    pltpu.async_copy(
                        feats_ref.at[pl.ds((b * C + cf + NBUF) * N, N)],
                        tbl.at[pl.ds(toff, N)], sem_t[kb])

                pltpu.async_copy(ob.at[pl.ds(ooff, MW * K)],
                                 gslice(3 + cf), sem_o[kb])
            return 0
        lax.fori_loop(0, C // NBUF, fgroup, 0)

        for kb in range(NBUF):
            cf = C - NBUF + kb
            pltpu.make_async_copy(
                ob.at[pl.ds(kb * MW * K, MW * K)],
                gslice(3 + cf),
                sem_o[kb]).wait()

    return body(xyzc, childc, feats)


def kernel(xyz, child_xyz, feats):
    xyzc = jnp.transpose(xyz, (0, 2, 1)).reshape(-1)
    childc = jnp.transpose(child_xyz, (0, 2, 1)).reshape(-1)
    grouped1, filled1 = _sc_query_group(xyzc, childc, feats.reshape(-1))
    grouped = grouped1.reshape(BS, COUT, M, K)
    filled = filled1.reshape(BS, M, K).astype(jnp.bool_)
    return grouped, filled


# speedup vs baseline: 701.1938x; 1.8617x over previous
import functools

import numpy as np
import jax
import jax.numpy as jnp
from jax import lax
from jax.experimental import pallas as pl
from jax.experimental.pallas import tpu as pltpu
from jax.experimental.pallas import tpu_sc as plsc

BS = 4
N = 8192
M = 2048
C = 128
COUT = C + 3
K = 32
MW = 256
NGROUP = MW // 16
G = 4
NV = MW * K // 16
R2 = np.float32(0.2 * 0.2)

SLOT = 8192
NSLOT = 13
CXOFF = NSLOT * SLOT
ARENA_WORDS = NSLOT * SLOT + 3 * MW

_DNUMS = lax.GatherDimensionNumbers(
    offset_dims=(), collapsed_slice_dims=(0,), start_index_map=(0,))


def _bcast(vec, t):
    return lax.gather(vec, jnp.full((16, 1), t, jnp.int32), _DNUMS,
                      slice_sizes=(1,),
                      mode=lax.GatherScatterMode.PROMISE_IN_BOUNDS)


def _sc_query_group(xyzc, childc, feats):
    mesh = plsc.VectorSubcoreMesh(core_axis_name="c", subcore_axis_name="s")

    @functools.partial(
        pl.kernel,
        out_type=(
            jax.ShapeDtypeStruct((BS * COUT * M * K,), jnp.float32),
            jax.ShapeDtypeStruct((BS * M * K,), jnp.int32),
        ),
        mesh=mesh,
        compiler_params=pltpu.CompilerParams(needs_layout_passes=False),
        scratch_types=[
            pltpu.VMEM((ARENA_WORDS,), jnp.float32),
            pltpu.VMEM((MW * K,), jnp.int32),
            pltpu.VMEM((MW * K,), jnp.int32),
            [pltpu.SemaphoreType.DMA] * G,
            [pltpu.SemaphoreType.DMA] * G,
        ],
    )
    def body(xyzc_ref, childc_ref, feats_ref, grouped_ref, filled_ref,
             arena, idxb, fillb, sem_t, sem_o):
        wid = lax.axis_index("s") * 2 + lax.axis_index("c")
        b = wid // 8
        mbase = (wid % 8) * MW
        obase = mbase * K

        def slot(s):
            return arena.at[pl.ds(s * SLOT, SLOT)]

        def oslot(q):
            return arena.at[pl.ds((9 + q) * SLOT, MW * K)]

        def gslice(ch):
            return grouped_ref.at[pl.ds((b * COUT + ch) * M * K + obase,
                                        MW * K)]

        def tsrc(ch):
            return feats_ref.at[pl.ds((b * C + ch) * N, N)]

        for d in range(3):
            pltpu.sync_copy(xyzc_ref.at[pl.ds((b * 3 + d) * N, N)], slot(d))
            pltpu.sync_copy(childc_ref.at[pl.ds((b * 3 + d) * M + mbase, MW)],
                            arena.at[pl.ds(CXOFF + d * MW, MW)])

        iota16 = lax.iota(jnp.int32, 16)
        zeros16 = jnp.zeros((16,), jnp.int32)

        def zstep(v, _):
            idxb[pl.ds(v * 16, 16)] = zeros16
            return 0
        lax.fori_loop(0, NV, zstep, 0)

        def group_body(g, _):
            base = g * 16
            cxv = arena[pl.ds(CXOFF + base, 16)]
            cyv = arena[pl.ds(CXOFF + MW + base, 16)]
            czv = arena[pl.ds(CXOFF + 2 * MW + base, 16)]
            rowbase = (base + iota16) * K

            def ocond(carry):
                j0, ptrv, done = carry
                return jnp.logical_and(j0 < N, jnp.logical_not(done))

            def obody(carry):
                j0, ptrv, _ = carry
                for u in range(2):
                    jc = j0 + u * 16
                    xc = arena[pl.ds(jc, 16)]
                    yc = arena[pl.ds(SLOT + jc, 16)]
                    zc = arena[pl.ds(2 * SLOT + jc, 16)]
                    jbase = jnp.full((16,), jc, jnp.int32)
                    for t in range(16):
                        dx = cxv - _bcast(xc, t)
                        dy = cyv - _bcast(yc, t)
                        dz = czv - _bcast(zc, t)
                        d2 = (dx * dx + dy * dy) + dz * dz
                        msk = d2 <= R2
                        okm = jnp.logical_and(msk, ptrv < K)
                        plsc.store_scatter(idxb, [rowbase + ptrv], jbase + t,
                                           mask=okm)
                        ptrv = ptrv + msk.astype(jnp.int32)
                ndone = jnp.sum((ptrv >= K).astype(jnp.int32))
                return (j0 + 32, ptrv, ndone >= 16)

            _, ptrv, _ = lax.while_loop(
                ocond, obody, (jnp.int32(0), zeros16, jnp.bool_(False)))
            cnt = jnp.minimum(ptrv, K)

            def fstep(s, _):
                plsc.store_scatter(fillb, [rowbase + s],
                                   (cnt > s).astype(jnp.int32))
                return 0
            lax.fori_loop(0, K, fstep, 0)
            return 0
        lax.fori_loop(0, NGROUP, group_body, 0)

        pltpu.sync_copy(fillb, filled_ref.at[pl.ds(b * M * K + obase, MW * K)])

        for q in range(G):
            pltpu.async_copy(tsrc(q), slot(4 + q), sem_t[q])

        def xstep(v, idxv):
            row = v // 2
            for q, coff in ((0, CXOFF), (1, CXOFF + MW), (2, CXOFF + 2 * MW)):
                cval = plsc.load_gather(
                    arena, [jnp.full((16,), coff + row, jnp.int32)])
                g = plsc.load_gather(slot(q), [idxv]) - cval
                arena[pl.ds((9 + q) * SLOT + v * 16, 16)] = g
            return idxb[pl.ds(((v + 1) & (NV - 1)) * 16, 16)]
        lax.fori_loop(0, NV, xstep, idxb[pl.ds(0, 16)])
        for q in range(3):
            pltpu.sync_copy(oslot(q), gslice(q))

        def dgroup(t2, _):
            for par, tbase in ((0, 4), (1, 0)):
                gi = 2 * t2 + par
                ch0 = gi * G
                for q in range(G):
                    pltpu.make_async_copy(tsrc(ch0 + q), slot(tbase + q),
                                          sem_t[q]).wait()
                nbase = 4 - tbase

                @pl.when(gi + 1 < C // G)
                def _():
                    for q in range(G):
                        pltpu.async_copy(tsrc(ch0 + G + q), slot(nbase + q),
                                         sem_t[q])

                @pl.when(gi >= 1)
                def _():
                    for q in range(G):
                        pltpu.make_async_copy(oslot(q),
                                              gslice(3 + ch0 - G + q),
                                              sem_o[q]).wait()

                def gbody(v, idxv, tbase=tbase):
                    gs = [plsc.load_gather(slot(tbase + q), [idxv])
                          for q in range(G)]
                    for q in range(G):
                        arena[pl.ds((9 + q) * SLOT + v * 16, 16)] = gs[q]
                    return idxb[pl.ds(((v + 1) & (NV - 1)) * 16, 16)]
                lax.fori_loop(0, NV, gbody, idxb[pl.ds(0, 16)], unroll=2)

                for q in range(G):
                    pltpu.async_copy(oslot(q), gslice(3 + ch0 + q), sem_o[q])
            return 0
        lax.fori_loop(0, C // (2 * G), dgroup, 0)

        for q in range(G):
            pltpu.make_async_copy(oslot(q), gslice(3 + C - G + q),
                                  sem_o[q]).wait()

    return body(xyzc, childc, feats)


def kernel(xyz, child_xyz, feats):
    xyzc = jnp.transpose(xyz, (0, 2, 1)).reshape(-1)
    childc = jnp.transpose(child_xyz, (0, 2, 1)).reshape(-1)
    grouped1, filled1 = _sc_query_group(xyzc, childc, feats.reshape(-1))
    grouped = grouped1.reshape(BS, COUT, M, K)
    filled = filled1.reshape(BS, M, K).astype(jnp.bool_)
    return grouped, filled


# speedup vs baseline: 746.5628x; 1.0647x over previous
import functools

import numpy as np
import jax
import jax.numpy as jnp
from jax import lax
from jax.experimental import pallas as pl
from jax.experimental.pallas import tpu as pltpu
from jax.experimental.pallas import tpu_sc as plsc

BS = 4
N = 8192
M = 2048
C = 128
COUT = C + 3
K = 32
MW = 256
NGROUP = MW // 16
G = 4
NV = MW * K // 16
R2 = np.float32(0.2 * 0.2)

SLOT = 8192
NSLOT = 13
CXOFF = NSLOT * SLOT
ARENA_WORDS = NSLOT * SLOT + 3 * MW

_DNUMS = lax.GatherDimensionNumbers(
    offset_dims=(), collapsed_slice_dims=(0,), start_index_map=(0,))


def _bcast(vec, t):
    return lax.gather(vec, jnp.full((16, 1), t, jnp.int32), _DNUMS,
                      slice_sizes=(1,),
                      mode=lax.GatherScatterMode.PROMISE_IN_BOUNDS)


def _sc_query_group(xyzc, childc, feats):
    mesh = plsc.VectorSubcoreMesh(core_axis_name="c", subcore_axis_name="s")

    @functools.partial(
        pl.kernel,
        out_type=(
            jax.ShapeDtypeStruct((BS * COUT * M * K,), jnp.float32),
            jax.ShapeDtypeStruct((BS * M * K,), jnp.int32),
        ),
        mesh=mesh,
        compiler_params=pltpu.CompilerParams(needs_layout_passes=False),
        scratch_types=[
            pltpu.VMEM((ARENA_WORDS,), jnp.float32),
            pltpu.VMEM((MW * K,), jnp.int32),
            pltpu.VMEM((MW * K,), jnp.int32),
            [pltpu.SemaphoreType.DMA] * G,
            [pltpu.SemaphoreType.DMA] * G,
        ],
    )
    def body(xyzc_ref, childc_ref, feats_ref, grouped_ref, filled_ref,
             arena, idxb, fillb, sem_t, sem_o):
        wid = lax.axis_index("s") * 2 + lax.axis_index("c")
        b = wid // 8
        mbase = (wid % 8) * MW
        obase = mbase * K

        def slot(s):
            return arena.at[pl.ds(s * SLOT, SLOT)]

        def oslot(q):
            return arena.at[pl.ds((9 + q) * SLOT, MW * K)]

        def gslice(ch):
            return grouped_ref.at[pl.ds((b * COUT + ch) * M * K + obase,
                                        MW * K)]

        def tsrc(ch):
            return feats_ref.at[pl.ds((b * C + ch) * N, N)]

        for d in range(3):
            pltpu.sync_copy(xyzc_ref.at[pl.ds((b * 3 + d) * N, N)], slot(d))
            pltpu.sync_copy(childc_ref.at[pl.ds((b * 3 + d) * M + mbase, MW)],
                            arena.at[pl.ds(CXOFF + d * MW, MW)])

        iota16 = lax.iota(jnp.int32, 16)
        zeros16 = jnp.zeros((16,), jnp.int32)

        def zstep(v, _):
            idxb[pl.ds(v * 16, 16)] = zeros16
            return 0
        lax.fori_loop(0, NV, zstep, 0)

        def group_body(g, _):
            base = g * 16
            cxv = arena[pl.ds(CXOFF + base, 16)]
            cyv = arena[pl.ds(CXOFF + MW + base, 16)]
            czv = arena[pl.ds(CXOFF + 2 * MW + base, 16)]
            rowbase = (base + iota16) * K

            def ocond(carry):
                j0, ptrv, done = carry
                return jnp.logical_and(j0 < N, jnp.logical_not(done))

            def obody(carry):
                j0, ptrv, _ = carry
                for u in range(2):
                    jc = j0 + u * 16
                    xc = arena[pl.ds(jc, 16)]
                    yc = arena[pl.ds(SLOT + jc, 16)]
                    zc = arena[pl.ds(2 * SLOT + jc, 16)]
                    jbase = jnp.full((16,), jc, jnp.int32)
                    for t in range(16):
                        dx = cxv - _bcast(xc, t)
                        dy = cyv - _bcast(yc, t)
                        dz = czv - _bcast(zc, t)
                        d2 = (dx * dx + dy * dy) + dz * dz
                        msk = d2 <= R2
                        okm = jnp.logical_and(msk, ptrv < K)
                        plsc.store_scatter(idxb, [rowbase + ptrv], jbase + t,
                                           mask=okm)
                        ptrv = ptrv + msk.astype(jnp.int32)
                ndone = jnp.sum((ptrv >= K).astype(jnp.int32))
                return (j0 + 32, ptrv, ndone >= 16)

            _, ptrv, _ = lax.while_loop(
                ocond, obody, (jnp.int32(0), zeros16, jnp.bool_(False)))
            cnt = jnp.minimum(ptrv, K)

            def fstep(s, _):
                plsc.store_scatter(fillb, [rowbase + s],
                                   (cnt > s).astype(jnp.int32))
                return 0
            lax.fori_loop(0, K, fstep, 0)
            return 0
        lax.fori_loop(0, NGROUP, group_body, 0)

        pltpu.sync_copy(fillb, filled_ref.at[pl.ds(b * M * K + obase, MW * K)])

        for q in range(G):
            pltpu.async_copy(tsrc(q), slot(4 + q), sem_t[q])

        def xstep(v, idxv):
            row = v // 2
            for q, coff in ((0, CXOFF), (1, CXOFF + MW), (2, CXOFF + 2 * MW)):
                cval = plsc.load_gather(
                    arena, [jnp.full((16,), coff + row, jnp.int32)])
                g = plsc.load_gather(slot(q), [idxv]) - cval
                arena[pl.ds((9 + q) * SLOT + v * 16, 16)] = g
            return idxb[pl.ds(((v + 1) & (NV - 1)) * 16, 16)]
        lax.fori_loop(0, NV, xstep, idxb[pl.ds(0, 16)])
        for q in range(3):
            pltpu.sync_copy(oslot(q), gslice(q))

        def dgroup(t2, _):
            for par, tbase in ((0, 4), (1, 0)):
                gi = 2 * t2 + par
                ch0 = gi * G
                for q in range(G):
                    pltpu.make_async_copy(tsrc(ch0 + q), slot(tbase + q),
                                          sem_t[q]).wait()
                nbase = 4 - tbase

                @pl.when(gi + 1 < C // G)
                def _():
                    for q in range(G):
                        pltpu.async_copy(tsrc(ch0 + G + q), slot(nbase + q),
                                         sem_t[q])

                @pl.when(gi >= 1)
                def _():
                    for q in range(G):
                        pltpu.make_async_copy(oslot(q),
                                              gslice(3 + ch0 - G + q),
                                              sem_o[q]).wait()

                def gbody(v, carry, tbase=tbase):
                    idxv, idxn = carry
                    gs = [plsc.load_gather(slot(tbase + q), [idxv])
                          for q in range(G)]
                    for q in range(G):
                        arena[pl.ds((9 + q) * SLOT + v * 16, 16)] = gs[q]
                    idx2 = idxb[pl.ds(((v + 2) & (NV - 1)) * 16, 16)]
                    return (idxn, idx2)
                lax.fori_loop(0, NV, gbody,
                              (idxb[pl.ds(0, 16)], idxb[pl.ds(16, 16)]),
                              unroll=2)

                for q in range(G):
                    pltpu.async_copy(oslot(q), gslice(3 + ch0 + q), sem_o[q])
            return 0
        lax.fori_loop(0, C // (2 * G), dgroup, 0)

        for q in range(G):
            pltpu.make_async_copy(oslot(q), gslice(3 + C - G + q),
                                  sem_o[q]).wait()

    return body(xyzc, childc, feats)


def kernel(xyz, child_xyz, feats):
    xyzc = jnp.transpose(xyz, (0, 2, 1)).reshape(-1)
    childc = jnp.transpose(child_xyz, (0, 2, 1)).reshape(-1)
    grouped1, filled1 = _sc_query_group(xyzc, childc, feats.reshape(-1))
    grouped = grouped1.reshape(BS, COUT, M, K)
    filled = filled1.reshape(BS, M, K).astype(jnp.bool_)
    return grouped, filled
